# baseline (speedup 1.0000x reference)
import functools

import jax
import jax.numpy as jnp
from jax import lax
from jax.experimental import pallas as pl
from jax.experimental.pallas import tpu as pltpu
from jax.experimental.pallas import tpu_sc as plsc

N = 10000
E = 320000
D = 128
NW = 32
K = 48
EPW = 10080
PADE = EPW - E // NW
CH = EPW // K
AGG_R = EPW
RPT = 640
TAIL = N - 15 * RPT
BLK = 1000


def _pre_body(x_ref, posp_ref, w1h_ref, b1h_ref, w2hp_ref, b2hp_ref,
              wf3p_ref, wfd_ref, bf_ref, a_ref, b_ref):
    x = x_ref[...]
    t = jnp.maximum(jnp.dot(x, w1h_ref[...],
                            preferred_element_type=jnp.float32)
                    + b1h_ref[...], 0.0)
    delta = jnp.tanh(jnp.dot(t, w2hp_ref[...],
                             preferred_element_type=jnp.float32)
                     + b2hp_ref[...])
    posw = jnp.dot(posp_ref[...], wf3p_ref[...],
                   preferred_element_type=jnp.float32)
    a_ref[...] = jnp.dot(x, wfd_ref[...],
                         preferred_element_type=jnp.float32) + posw
    b_ref[...] = (jnp.dot(delta, wf3p_ref[...],
                          preferred_element_type=jnp.float32)
                  - posw + bf_ref[...])


def _run_pre(x, pos_pad, W1h, b1h, W2h_pad, b2h_pad, Wf3_pad, WfD, bf):
    row_spec = pl.BlockSpec((BLK, D), lambda i: (i, 0))
    w_spec = pl.BlockSpec((D, D), lambda i: (0, 0))
    bias_spec = pl.BlockSpec((1, D), lambda i: (0, 0))
    return pl.pallas_call(
        _pre_body,
        grid=(N // BLK,),
        in_specs=[row_spec, row_spec, w_spec, bias_spec, w_spec, bias_spec,
                  w_spec, w_spec, bias_spec],
        out_specs=[row_spec, row_spec],
        out_shape=[jax.ShapeDtypeStruct((N, D), jnp.float32),
                   jax.ShapeDtypeStruct((N, D), jnp.float32)],
    )(x, pos_pad, W1h, b1h, W2h_pad, b2h_pad, Wf3_pad, WfD, bf)


def _edge_body(a_hbm, b_hbm, ei_hbm, out_hbm,
               eidx, arows0, arows1, arows2, brows0, brows1, aggsh,
               sema0, sema1, sema2, semb0, semb1, semi0, semi1,
               sems0, sems1, sems2):
    cid = lax.axis_index("c")
    sid = lax.axis_index("s")
    wid = sid * 2 + cid
    abuf = (arows0, arows1, arows2)
    bbuf = (brows0, brows1)
    asem = (sema0, sema1, sema2)
    bsem = (semb0, semb1)
    isem = (semi0, semi1)
    ssem = (sems0, sems1, sems2)

    zero16 = jnp.zeros((16,), jnp.float32)

    def zrow(i, carry):
        for j in range(8):
            arows0[i, pl.ds(j * 16, 16)] = zero16
        return carry

    lax.fori_loop(0, K, zrow, 0)
    nz = jnp.where(sid < 2, 14, 13)

    def zcopy(r, carry):
        z = sid + 16 * r
        pltpu.sync_copy(arows0, aggsh.at[pl.ds(z * K, K)])
        return carry

    lax.fori_loop(0, nz, zcopy, 0)
    plsc.subcore_barrier()

    def start_idx(c, p2):
        base = wid * EPW + c * K
        pltpu.async_copy(ei_hbm.at[pl.ds(base, K)], eidx.at[p2], isem[p2])
        pltpu.async_copy(ei_hbm.at[pl.ds(NW * EPW + base, K)],
                         eidx.at[2 + p2], isem[p2])

    def wait_idx(c, p2):
        base = wid * EPW + c * K
        pltpu.make_async_copy(ei_hbm.at[pl.ds(base, K)], eidx.at[p2],
                              isem[p2]).wait()
        pltpu.make_async_copy(ei_hbm.at[pl.ds(NW * EPW + base, K)],
                              eidx.at[2 + p2], isem[p2]).wait()

    def start_rows(p3, p2):
        pltpu.async_copy(a_hbm.at[eidx.at[p2]], abuf[p3], asem[p3])
        pltpu.async_copy(b_hbm.at[eidx.at[2 + p2]], bbuf[p2], bsem[p2])

    def wait_rows(p3, p2):
        pltpu.make_async_copy(a_hbm.at[eidx.at[p2]], abuf[p3],
                              asem[p3]).wait()
        pltpu.make_async_copy(b_hbm.at[eidx.at[2 + p2]], bbuf[p2],
                              bsem[p2]).wait()

    def wait_scatter(p3):
        pltpu.make_async_copy(abuf[p3], aggsh.at[eidx.at[4 + p3]],
                              ssem[p3]).wait()

    def step(c, k):
        p3, p2 = k % 3, k % 2
        q3, q2 = (k + 1) % 3, (k + 1) % 2

        @pl.when(c + 1 < CH)
        def _():
            wait_idx(c + 1, q2)

            @pl.when(c + 1 >= 3)
            def _():
                wait_scatter(q3)

            start_rows(q3, q2)

        wait_rows(p3, p2)
        for m in range(K // 16):
            eidx[4 + p3, pl.ds(m * 16, 16)] = eidx[2 + p2, pl.ds(m * 16, 16)]

        @pl.when(c + 2 < CH)
        def _():
            start_idx(c + 2, p2)

        ar, br = abuf[p3], bbuf[p2]

        def row(i, rc):
            r0 = 2 * i
            for r in range(2):
                for j in range(8):
                    sl = pl.ds(j * 16, 16)
                    ar[r0 + r, sl] = jnp.maximum(
                        ar[r0 + r, sl] + br[r0 + r, sl], 0.0)
            return rc

        lax.fori_loop(0, K // 2, row, 0)
        pltpu.async_copy(ar, aggsh.at[eidx.at[4 + p3]], ssem[p3], add=True)

    start_idx(0, 0)
    wait_idx(0, 0)
    start_rows(0, 0)
    start_idx(1, 1)

    def six(i, carry):
        c0 = 6 * i
        for k in range(6):
            step(c0 + k, k)
        return carry

    lax.fori_loop(0, CH // 6, six, 0)
    wait_scatter(0)
    wait_scatter(1)
    wait_scatter(2)
    plsc.subcore_barrier()

    @pl.when(sid < 15)
    def _():
        pltpu.sync_copy(aggsh.at[pl.ds(sid * RPT, RPT)],
                        out_hbm.at[cid, pl.ds(sid * RPT, RPT)])

    @pl.when(sid == 15)
    def _():
        pltpu.sync_copy(aggsh.at[pl.ds(15 * RPT, TAIL)],
                        out_hbm.at[cid, pl.ds(15 * RPT, TAIL)])


_edge_kernel = functools.partial(
    pl.kernel,
    out_type=jax.ShapeDtypeStruct((2, N, D), jnp.float32),
    mesh=plsc.VectorSubcoreMesh(core_axis_name="c", subcore_axis_name="s"),
    scratch_types=[
        pltpu.VMEM((8, K), jnp.int32),
        pltpu.VMEM((K, D), jnp.float32),
        pltpu.VMEM((K, D), jnp.float32),
        pltpu.VMEM((K, D), jnp.float32),
        pltpu.VMEM((K, D), jnp.float32),
        pltpu.VMEM((K, D), jnp.float32),
        pltpu.VMEM_SHARED((AGG_R, D), jnp.float32),
    ] + [pltpu.SemaphoreType.DMA] * 10,
)(_edge_body)


def _post_body(agg0_ref, agg1_ref, x_ref, w1g_ref, b1g_ref, w2g_ref,
               b2g_ref, out_ref):
    agg = agg0_ref[...] + agg1_ref[...]
    h = jnp.maximum(jnp.dot(agg, w1g_ref[...],
                            preferred_element_type=jnp.float32)
                    + b1g_ref[...], 0.0)
    out_ref[...] = x_ref[...] + jnp.maximum(
        jnp.dot(h, w2g_ref[...], preferred_element_type=jnp.float32)
        + b2g_ref[...], 0.0)


def _run_post(agg0, agg1, x, W1g, b1g, W2g, b2g):
    row_spec = pl.BlockSpec((BLK, D), lambda i: (i, 0))
    w_spec = pl.BlockSpec((D, D), lambda i: (0, 0))
    bias_spec = pl.BlockSpec((1, D), lambda i: (0, 0))
    return pl.pallas_call(
        _post_body,
        grid=(N // BLK,),
        in_specs=[row_spec, row_spec, row_spec, w_spec, bias_spec, w_spec,
                  bias_spec],
        out_specs=row_spec,
        out_shape=jax.ShapeDtypeStruct((N, D), jnp.float32),
    )(agg0, agg1, x, W1g, b1g, W2g, b2g)


def kernel(x, pos, edge_index, W1h, b1h, W2h, b2h, Wf, bf, W1g, b1g, W2g,
           b2g):
    f32 = jnp.float32
    pos_pad = jnp.pad(pos, ((0, 0), (0, D - 3)))
    W2h_pad = jnp.pad(W2h, ((0, 0), (0, D - 3)))
    b2h_pad = jnp.pad(b2h, (0, D - 3)).reshape(1, D)
    Wf3_pad = jnp.pad(Wf[:3], ((0, D - 3), (0, 0)))
    WfD = Wf[3:]
    a_nodes, b_nodes = _run_pre(x, pos_pad, W1h, b1h.reshape(1, D).astype(f32),
                                W2h_pad, b2h_pad.astype(f32), Wf3_pad, WfD,
                                bf.reshape(1, D).astype(f32))
    a_tab = jnp.pad(a_nodes, ((0, 8), (0, 0)))
    b_tab = jnp.pad(b_nodes, ((0, 8), (0, 0)))
    pad_cols = jnp.full((NW, PADE), N, dtype=jnp.int32)
    ei_pad = jnp.concatenate(
        [edge_index.reshape(2, NW, E // NW),
         jnp.broadcast_to(pad_cols, (2, NW, PADE))], axis=2)
    agg2 = _edge_kernel(a_tab, b_tab, ei_pad.reshape(2 * NW * EPW))
    return _run_post(agg2[0], agg2[1], x, W1g,
                     b1g.reshape(1, D).astype(f32), W2g,
                     b2g.reshape(1, D).astype(f32))


# speedup vs baseline: 1.1333x; 1.1333x over previous
import functools

import jax
import jax.numpy as jnp
from jax import lax
from jax.experimental import pallas as pl
from jax.experimental.pallas import tpu as pltpu
from jax.experimental.pallas import tpu_sc as plsc

N = 10000
E = 320000
D = 128
NW = 32
K = 64
EPW = 10048
PADE = EPW - E // NW
CH = EPW // K
AGG_R = EPW
RPT = 640
TAIL = N - 15 * RPT
BLK = 1000


def _pre_body(x_ref, posp_ref, w1h_ref, b1h_ref, w2hp_ref, b2hp_ref,
              wf3p_ref, wfd_ref, bf_ref, a_ref, b_ref):
    x = x_ref[...]
    t = jnp.maximum(jnp.dot(x, w1h_ref[...],
                            preferred_element_type=jnp.float32)
                    + b1h_ref[...], 0.0)
    delta = jnp.tanh(jnp.dot(t, w2hp_ref[...],
                             preferred_element_type=jnp.float32)
                     + b2hp_ref[...])
    posw = jnp.dot(posp_ref[...], wf3p_ref[...],
                   preferred_element_type=jnp.float32)
    a_ref[...] = jnp.dot(x, wfd_ref[...],
                         preferred_element_type=jnp.float32) + posw
    b_ref[...] = (jnp.dot(delta, wf3p_ref[...],
                          preferred_element_type=jnp.float32)
                  - posw + bf_ref[...])


def _run_pre(x, pos_pad, W1h, b1h, W2h_pad, b2h_pad, Wf3_pad, WfD, bf):
    row_spec = pl.BlockSpec((BLK, D), lambda i: (i, 0))
    w_spec = pl.BlockSpec((D, D), lambda i: (0, 0))
    bias_spec = pl.BlockSpec((1, D), lambda i: (0, 0))
    return pl.pallas_call(
        _pre_body,
        grid=(N // BLK,),
        in_specs=[row_spec, row_spec, w_spec, bias_spec, w_spec, bias_spec,
                  w_spec, w_spec, bias_spec],
        out_specs=[row_spec, row_spec],
        out_shape=[jax.ShapeDtypeStruct((N, D), jnp.float32),
                   jax.ShapeDtypeStruct((N, D), jnp.float32)],
    )(x, pos_pad, W1h, b1h, W2h_pad, b2h_pad, Wf3_pad, WfD, bf)


def _edge_body(a_hbm, b_hbm, ei_hbm, out_hbm,
               eidx, arows0, brows0, arows1, brows1, aggsh,
               sema0, sema1, semb0, semb1, semi0, semi1):
    cid = lax.axis_index("c")
    sid = lax.axis_index("s")
    wid = sid * 2 + cid
    abuf = (arows0, arows1)
    bbuf = (brows0, brows1)
    asem = (sema0, sema1)
    bsem = (semb0, semb1)
    isem = (semi0, semi1)

    zero16 = jnp.zeros((16,), jnp.float32)

    def zrow(i, carry):
        for j in range(8):
            arows0[i, pl.ds(j * 16, 16)] = zero16
        return carry

    lax.fori_loop(0, K, zrow, 0)
    nz = jnp.where(sid < 13, 10, 9)

    def zcopy(r, carry):
        z = sid + 16 * r
        pltpu.sync_copy(arows0, aggsh.at[pl.ds(z * K, K)])
        return carry

    lax.fori_loop(0, nz, zcopy, 0)
    plsc.subcore_barrier()

    def start_idx(c, b):
        base = wid * EPW + c * K
        pltpu.async_copy(ei_hbm.at[pl.ds(base, K)], eidx.at[b], isem[b])
        pltpu.async_copy(ei_hbm.at[pl.ds(NW * EPW + base, K)],
                         eidx.at[2 + b], isem[b])

    def wait_idx(c, b):
        base = wid * EPW + c * K
        pltpu.make_async_copy(ei_hbm.at[pl.ds(base, K)], eidx.at[b],
                              isem[b]).wait()
        pltpu.make_async_copy(ei_hbm.at[pl.ds(NW * EPW + base, K)],
                              eidx.at[2 + b], isem[b]).wait()

    def start_rows(b):
        pltpu.async_copy(a_hbm.at[eidx.at[b]], abuf[b], asem[b])
        pltpu.async_copy(b_hbm.at[eidx.at[2 + b]], bbuf[b], bsem[b])

    def wait_rows(b):
        pltpu.make_async_copy(a_hbm.at[eidx.at[b]], abuf[b], asem[b]).wait()
        pltpu.make_async_copy(b_hbm.at[eidx.at[2 + b]], bbuf[b],
                              bsem[b]).wait()

    def snapshot(b):
        for m in range(K // 16):
            eidx[4 + b, pl.ds(m * 16, 16)] = eidx[2 + b, pl.ds(m * 16, 16)]

    def process(b):
        ar, br = abuf[b], bbuf[b]

        def row(i, rc):
            r0 = 2 * i
            for r in range(2):
                for j in range(8):
                    sl = pl.ds(j * 16, 16)
                    ar[r0 + r, sl] = jnp.maximum(
                        ar[r0 + r, sl] + br[r0 + r, sl], 0.0)
            return rc

        lax.fori_loop(0, K // 2, row, 0)
        pltpu.sync_copy(ar, aggsh.at[eidx.at[4 + b]], add=True)

    start_idx(0, 0)
    wait_idx(0, 0)
    start_rows(0)
    start_idx(1, 1)

    def pair(i, carry):
        c0 = 2 * i
        wait_idx(c0 + 1, 1)
        start_rows(1)
        wait_rows(0)
        snapshot(0)
        start_idx(c0 + 2, 0)
        process(0)
        wait_idx(c0 + 2, 0)
        start_rows(0)
        wait_rows(1)
        snapshot(1)

        @pl.when(c0 + 3 < CH)
        def _():
            start_idx(c0 + 3, 1)

        process(1)
        return carry

    lax.fori_loop(0, CH // 2, pair, 0)
    wait_rows(0)
    snapshot(0)
    process(0)
    plsc.subcore_barrier()

    @pl.when(sid < 15)
    def _():
        pltpu.sync_copy(aggsh.at[pl.ds(sid * RPT, RPT)],
                        out_hbm.at[cid, pl.ds(sid * RPT, RPT)])

    @pl.when(sid == 15)
    def _():
        pltpu.sync_copy(aggsh.at[pl.ds(15 * RPT, TAIL)],
                        out_hbm.at[cid, pl.ds(15 * RPT, TAIL)])


_edge_kernel = functools.partial(
    pl.kernel,
    out_type=jax.ShapeDtypeStruct((2, N, D), jnp.float32),
    mesh=plsc.VectorSubcoreMesh(core_axis_name="c", subcore_axis_name="s"),
    scratch_types=[
        pltpu.VMEM((6, K), jnp.int32),
        pltpu.VMEM((K, D), jnp.float32),
        pltpu.VMEM((K, D), jnp.float32),
        pltpu.VMEM((K, D), jnp.float32),
        pltpu.VMEM((K, D), jnp.float32),
        pltpu.VMEM_SHARED((AGG_R, D), jnp.float32),
        pltpu.SemaphoreType.DMA,
        pltpu.SemaphoreType.DMA,
        pltpu.SemaphoreType.DMA,
        pltpu.SemaphoreType.DMA,
        pltpu.SemaphoreType.DMA,
        pltpu.SemaphoreType.DMA,
    ],
)(_edge_body)


def _post_body(agg0_ref, agg1_ref, x_ref, w1g_ref, b1g_ref, w2g_ref,
               b2g_ref, out_ref):
    agg = agg0_ref[...] + agg1_ref[...]
    h = jnp.maximum(jnp.dot(agg, w1g_ref[...],
                            preferred_element_type=jnp.float32)
                    + b1g_ref[...], 0.0)
    out_ref[...] = x_ref[...] + jnp.maximum(
        jnp.dot(h, w2g_ref[...], preferred_element_type=jnp.float32)
        + b2g_ref[...], 0.0)


def _run_post(agg0, agg1, x, W1g, b1g, W2g, b2g):
    row_spec = pl.BlockSpec((BLK, D), lambda i: (i, 0))
    w_spec = pl.BlockSpec((D, D), lambda i: (0, 0))
    bias_spec = pl.BlockSpec((1, D), lambda i: (0, 0))
    return pl.pallas_call(
        _post_body,
        grid=(N // BLK,),
        in_specs=[row_spec, row_spec, row_spec, w_spec, bias_spec, w_spec,
                  bias_spec],
        out_specs=row_spec,
        out_shape=jax.ShapeDtypeStruct((N, D), jnp.float32),
    )(agg0, agg1, x, W1g, b1g, W2g, b2g)


def kernel(x, pos, edge_index, W1h, b1h, W2h, b2h, Wf, bf, W1g, b1g, W2g,
           b2g):
    f32 = jnp.float32
    pos_pad = jnp.pad(pos, ((0, 0), (0, D - 3)))
    W2h_pad = jnp.pad(W2h, ((0, 0), (0, D - 3)))
    b2h_pad = jnp.pad(b2h, (0, D - 3)).reshape(1, D)
    Wf3_pad = jnp.pad(Wf[:3], ((0, D - 3), (0, 0)))
    WfD = Wf[3:]
    a_nodes, b_nodes = _run_pre(x, pos_pad, W1h, b1h.reshape(1, D).astype(f32),
                                W2h_pad, b2h_pad.astype(f32), Wf3_pad, WfD,
                                bf.reshape(1, D).astype(f32))
    a_tab = jnp.pad(a_nodes, ((0, 8), (0, 0)))
    b_tab = jnp.pad(b_nodes, ((0, 8), (0, 0)))
    pad_cols = jnp.full((NW, PADE), N, dtype=jnp.int32)
    ei_pad = jnp.concatenate(
        [edge_index.reshape(2, NW, E // NW),
         jnp.broadcast_to(pad_cols, (2, NW, PADE))], axis=2)
    agg2 = _edge_kernel(a_tab, b_tab, ei_pad.reshape(2 * NW * EPW))
    return _run_post(agg2[0], agg2[1], x, W1g,
                     b1g.reshape(1, D).astype(f32), W2g,
                     b2g.reshape(1, D).astype(f32))


# speedup vs baseline: 1.4924x; 1.3169x over previous
import functools

import jax
import jax.numpy as jnp
from jax import lax
from jax.experimental import pallas as pl
from jax.experimental.pallas import tpu as pltpu
from jax.experimental.pallas import tpu_sc as plsc

N = 10000
E = 320000
D = 128
NW = 32
K = 80
EPW = E // NW
CH = EPW // K
AGG_R = N
RPT = 640
TAIL = N - 15 * RPT
BLK = 2000


def _pre_body(x_ref, posp_ref, w1h_ref, b1h_ref, w2hp_ref, b2hp_ref,
              wf3p_ref, wfd_ref, bf_ref, a_ref, b_ref):
    x = x_ref[...]
    t = jnp.maximum(jnp.dot(x, w1h_ref[...],
                            preferred_element_type=jnp.float32)
                    + b1h_ref[...], 0.0)
    delta = jnp.tanh(jnp.dot(t, w2hp_ref[...],
                             preferred_element_type=jnp.float32)
                     + b2hp_ref[...])
    posw = jnp.dot(posp_ref[...], wf3p_ref[...],
                   preferred_element_type=jnp.float32)
    a_ref[...] = jnp.dot(x, wfd_ref[...],
                         preferred_element_type=jnp.float32) + posw
    b_ref[...] = (jnp.dot(delta, wf3p_ref[...],
                          preferred_element_type=jnp.float32)
                  - posw + bf_ref[...])


def _run_pre(x, pos_pad, W1h, b1h, W2h_pad, b2h_pad, Wf3_pad, WfD, bf):
    row_spec = pl.BlockSpec((BLK, D), lambda i: (i, 0))
    w_spec = pl.BlockSpec((D, D), lambda i: (0, 0))
    bias_spec = pl.BlockSpec((1, D), lambda i: (0, 0))
    return pl.pallas_call(
        _pre_body,
        grid=(N // BLK,),
        in_specs=[row_spec, row_spec, w_spec, bias_spec, w_spec, bias_spec,
                  w_spec, w_spec, bias_spec],
        out_specs=[row_spec, row_spec],
        out_shape=[jax.ShapeDtypeStruct((N, D), jnp.float32),
                   jax.ShapeDtypeStruct((N, D), jnp.float32)],
    )(x, pos_pad, W1h, b1h, W2h_pad, b2h_pad, Wf3_pad, WfD, bf)


def _edge_body(a_hbm, b_hbm, ei_hbm, out_hbm,
               eidx, arows0, brows0, arows1, brows1, aggsh,
               sema0, sema1, semb0, semb1, semi0, semi1):
    cid = lax.axis_index("c")
    sid = lax.axis_index("s")
    wid = sid * 2 + cid
    abuf = (arows0, arows1)
    bbuf = (brows0, brows1)
    asem = (sema0, sema1)
    bsem = (semb0, semb1)
    isem = (semi0, semi1)

    zero16 = jnp.zeros((16,), jnp.float32)

    def zrow(i, carry):
        for j in range(8):
            arows0[i, pl.ds(j * 16, 16)] = zero16
        return carry

    lax.fori_loop(0, K, zrow, 0)
    nz = jnp.where(sid == 15, TAIL // K, RPT // K)

    def zcopy(r, carry):
        pltpu.sync_copy(arows0, aggsh.at[pl.ds(sid * RPT + r * K, K)])
        return carry

    lax.fori_loop(0, nz, zcopy, 0)
    plsc.subcore_barrier()

    def start_idx(c, b):
        base = wid * EPW + c * K
        pltpu.async_copy(ei_hbm.at[pl.ds(base, K)], eidx.at[b], isem[b])
        pltpu.async_copy(ei_hbm.at[pl.ds(NW * EPW + base, K)],
                         eidx.at[2 + b], isem[b])

    def wait_idx(c, b):
        base = wid * EPW + c * K
        pltpu.make_async_copy(ei_hbm.at[pl.ds(base, K)], eidx.at[b],
                              isem[b]).wait()
        pltpu.make_async_copy(ei_hbm.at[pl.ds(NW * EPW + base, K)],
                              eidx.at[2 + b], isem[b]).wait()

    def start_rows(b):
        pltpu.async_copy(a_hbm.at[eidx.at[b]], abuf[b], asem[b])
        pltpu.async_copy(b_hbm.at[eidx.at[2 + b]], bbuf[b], bsem[b])

    def wait_rows(b):
        pltpu.make_async_copy(a_hbm.at[eidx.at[b]], abuf[b], asem[b]).wait()
        pltpu.make_async_copy(b_hbm.at[eidx.at[2 + b]], bbuf[b],
                              bsem[b]).wait()

    def snapshot(b):
        for m in range(K // 16):
            eidx[4 + b, pl.ds(m * 16, 16)] = eidx[2 + b, pl.ds(m * 16, 16)]

    def process(b):
        ar, br = abuf[b], bbuf[b]

        def row(i, rc):
            r0 = 4 * i
            for r in range(4):
                for j in range(8):
                    sl = pl.ds(j * 16, 16)
                    ar[r0 + r, sl] = jnp.maximum(
                        ar[r0 + r, sl] + br[r0 + r, sl], 0.0)
            return rc

        lax.fori_loop(0, K // 4, row, 0)
        pltpu.sync_copy(ar, aggsh.at[eidx.at[4 + b]], add=True)

    start_idx(0, 0)
    wait_idx(0, 0)
    start_rows(0)
    start_idx(1, 1)

    def pair(i, carry):
        c0 = 2 * i
        wait_idx(c0 + 1, 1)
        start_rows(1)
        wait_rows(0)
        snapshot(0)
        start_idx(c0 + 2, 0)
        process(0)
        wait_idx(c0 + 2, 0)
        start_rows(0)
        wait_rows(1)
        snapshot(1)

        @pl.when(c0 + 3 < CH)
        def _():
            start_idx(c0 + 3, 1)

        process(1)
        return carry

    lax.fori_loop(0, CH // 2, pair, 0)
    wait_rows(0)
    snapshot(0)
    process(0)
    plsc.subcore_barrier()

    @pl.when(sid < 15)
    def _():
        pltpu.sync_copy(aggsh.at[pl.ds(sid * RPT, RPT)],
                        out_hbm.at[cid, pl.ds(sid * RPT, RPT)])

    @pl.when(sid == 15)
    def _():
        pltpu.sync_copy(aggsh.at[pl.ds(15 * RPT, TAIL)],
                        out_hbm.at[cid, pl.ds(15 * RPT, TAIL)])


_edge_kernel = functools.partial(
    pl.kernel,
    out_type=jax.ShapeDtypeStruct((2, N, D), jnp.float32),
    mesh=plsc.VectorSubcoreMesh(core_axis_name="c", subcore_axis_name="s"),
    scratch_types=[
        pltpu.VMEM((6, K), jnp.int32),
        pltpu.VMEM((K, D), jnp.float32),
        pltpu.VMEM((K, D), jnp.float32),
        pltpu.VMEM((K, D), jnp.float32),
        pltpu.VMEM((K, D), jnp.float32),
        pltpu.VMEM_SHARED((AGG_R, D), jnp.float32),
        pltpu.SemaphoreType.DMA,
        pltpu.SemaphoreType.DMA,
        pltpu.SemaphoreType.DMA,
        pltpu.SemaphoreType.DMA,
        pltpu.SemaphoreType.DMA,
        pltpu.SemaphoreType.DMA,
    ],
)(_edge_body)


def _post_body(agg0_ref, agg1_ref, x_ref, w1g_ref, b1g_ref, w2g_ref,
               b2g_ref, out_ref):
    agg = agg0_ref[...] + agg1_ref[...]
    h = jnp.maximum(jnp.dot(agg, w1g_ref[...],
                            preferred_element_type=jnp.float32)
                    + b1g_ref[...], 0.0)
    out_ref[...] = x_ref[...] + jnp.maximum(
        jnp.dot(h, w2g_ref[...], preferred_element_type=jnp.float32)
        + b2g_ref[...], 0.0)


def _run_post(agg0, agg1, x, W1g, b1g, W2g, b2g):
    row_spec = pl.BlockSpec((BLK, D), lambda i: (i, 0))
    w_spec = pl.BlockSpec((D, D), lambda i: (0, 0))
    bias_spec = pl.BlockSpec((1, D), lambda i: (0, 0))
    return pl.pallas_call(
        _post_body,
        grid=(N // BLK,),
        in_specs=[row_spec, row_spec, row_spec, w_spec, bias_spec, w_spec,
                  bias_spec],
        out_specs=row_spec,
        out_shape=jax.ShapeDtypeStruct((N, D), jnp.float32),
    )(agg0, agg1, x, W1g, b1g, W2g, b2g)


def kernel(x, pos, edge_index, W1h, b1h, W2h, b2h, Wf, bf, W1g, b1g, W2g,
           b2g):
    f32 = jnp.float32
    pos_pad = jnp.pad(pos, ((0, 0), (0, D - 3)))
    W2h_pad = jnp.pad(W2h, ((0, 0), (0, D - 3)))
    b2h_pad = jnp.pad(b2h, (0, D - 3)).reshape(1, D)
    Wf3_pad = jnp.pad(Wf[:3], ((0, D - 3), (0, 0)))
    WfD = Wf[3:]
    a_nodes, b_nodes = _run_pre(x, pos_pad, W1h, b1h.reshape(1, D).astype(f32),
                                W2h_pad, b2h_pad.astype(f32), Wf3_pad, WfD,
                                bf.reshape(1, D).astype(f32))
    agg2 = _edge_kernel(a_nodes, b_nodes, edge_index.reshape(2 * E))
    return _run_post(agg2[0], agg2[1], x, W1g,
                     b1g.reshape(1, D).astype(f32), W2g,
                     b2g.reshape(1, D).astype(f32))
